# Initial kernel scaffold; baseline (speedup 1.0000x reference)
import jax
import jax.numpy as jnp
from jax.experimental import pallas as pl


def kernel(h, edge_index, e, graph_ids, atom_emb, Ws, bs, gammas, betas, mlp_W0, mlp_b0, mlp_W1, mlp_b1, mlp_W2, mlp_b2):
    raise NotImplementedError("write your pallas kernel here")


# speedup vs baseline: 4.1993x; 4.1993x over previous
import functools

import jax
import jax.numpy as jnp
from jax import lax
from jax.experimental import pallas as pl
from jax.experimental.pallas import tpu as pltpu
from jax.experimental.pallas import tpu_sc as plsc

_N = 10000
_E = 320000
_H = 128
_L = 4
_NF = 9
_NG = 128

_NP = 10240
_NW = 32
_DUMMY = 10000
_CE_L = 79
_CE_E = 23
_BN = 2048
_GRID = _NP // _BN

_f32 = jnp.float32
_i32 = jnp.int32


def _pad_edges(src, dst, nchunks, src_pad):
    total = _NW * nchunks * 128
    pad = total - src.shape[0]
    src = jnp.concatenate([src.astype(_i32), jnp.full((pad,), src_pad, _i32)])
    dst = jnp.concatenate([dst.astype(_i32), jnp.full((pad,), _DUMMY, _i32)])
    return src.reshape(_NW, nchunks, 128), dst.reshape(_NW, nchunks, 128)


def _make_sc_scatter(nchunks):
    mesh = plsc.VectorSubcoreMesh(core_axis_name="c", subcore_axis_name="s")

    @functools.partial(
        pl.kernel,
        out_type=jax.ShapeDtypeStruct((2, _NP, _H), _f32),
        mesh=mesh,
        scratch_types=[
            pltpu.VMEM((nchunks, 128), _i32),
            pltpu.VMEM((nchunks, 128), _i32),
            pltpu.VMEM((128, _H), _f32),
            pltpu.VMEM_SHARED((_NP, _H), _f32),
            pltpu.SemaphoreType.DMA,
        ],
    )
    def sc_scatter(table, src3, dst3, out, srcv, dstv, rows, agg_sh, sem):
        c = lax.axis_index("c")
        s = lax.axis_index("s")
        tid = c * 16 + s

        def zrow(i, _):
            for q in range(8):
                rows[i, pl.ds(q * 16, 16)] = jnp.zeros((16,), _f32)
            return 0

        lax.fori_loop(0, 128, zrow, 0)
        for k in range(_NP // (16 * 128)):
            pltpu.sync_copy(rows, agg_sh.at[pl.ds(s * 640 + k * 128, 128)])
        plsc.subcore_barrier()

        pltpu.sync_copy(src3.at[tid], srcv)
        pltpu.sync_copy(dst3.at[tid], dstv)

        def step(j, _):
            pltpu.async_copy(table.at[srcv.at[j]], rows, sem).wait()
            pltpu.sync_copy(rows, agg_sh.at[dstv.at[j]], add=True)
            return 0

        lax.fori_loop(0, nchunks, step, 0)
        plsc.subcore_barrier()
        pltpu.sync_copy(agg_sh.at[pl.ds(s * 640, 640)],
                        out.at[c, pl.ds(s * 640, 640)])

    return sc_scatter


def _make_sc_degrees():
    mesh = plsc.VectorSubcoreMesh(core_axis_name="c", subcore_axis_name="s")
    nbins = 2 * _NP

    @functools.partial(
        pl.kernel,
        out_type=jax.ShapeDtypeStruct((_NW, nbins), _f32),
        mesh=mesh,
        scratch_types=[
            pltpu.VMEM((_CE_L, 128), _i32),
            pltpu.VMEM((_CE_L, 128), _i32),
            pltpu.VMEM((nbins,), _f32),
        ],
        compiler_params=pltpu.CompilerParams(needs_layout_passes=False),
    )
    def sc_deg(src3, dst3, out, srcv, dstv, degv):
        c = lax.axis_index("c")
        s = lax.axis_index("s")
        tid = c * 16 + s

        def zrow(i, _):
            degv[pl.ds(i * 16, 16)] = jnp.zeros((16,), _f32)
            return 0

        lax.fori_loop(0, nbins // 16, zrow, 0)
        pltpu.sync_copy(src3.at[tid], srcv)
        pltpu.sync_copy(dst3.at[tid], dstv)
        ones = jnp.ones((16,), _f32)

        def step(j, _):
            for q in range(8):
                a = srcv[j, pl.ds(q * 16, 16)]
                plsc.addupdate_scatter(degv, [a], ones)
                b = dstv[j, pl.ds(q * 16, 16)] + _NP
                plsc.addupdate_scatter(degv, [b], ones)
            return 0

        lax.fori_loop(0, _CE_L, step, 0)
        pltpu.sync_copy(degv, out.at[tid])

    return sc_deg


def _tc_norms(degp):
    def body(d_ref, n_ref):
        d = jnp.sum(d_ref[...], axis=0)
        n_ref[...] = 1.0 / jnp.sqrt(jnp.maximum(d, 1.0))

    nrows = 2 * _NP // 128
    return pl.pallas_call(
        body,
        out_shape=jax.ShapeDtypeStruct((nrows, 128), _f32),
    )(degp)


def _expand_lanes(lm):
    r = lax.broadcasted_iota(_i32, (128, 128), 0)
    c = lax.broadcasted_iota(_i32, (128, 128), 1)
    ident = (r == c).astype(_f32)
    parts = []
    for k in range(16):
        col = lax.dot_general(ident, lm[k:k + 1, :], (((1,), (1,)), ((), ())),
                              precision=lax.Precision.HIGHEST,
                              preferred_element_type=_f32)
        parts.append(col * jnp.ones((1, _H), _f32))
    return jnp.concatenate(parts, axis=0)


def _tc_pass0(xp, norms_lm):
    def body(xp_ref, ns_ref, nd_ref, x_ref, xn_ref, nsb_ref, ndb_ref):
        nsb = _expand_lanes(ns_ref[...])
        ndb = _expand_lanes(nd_ref[...])
        x = xp_ref[0] + xp_ref[1]
        x_ref[...] = x
        xn_ref[...] = x * nsb
        nsb_ref[...] = nsb
        ndb_ref[...] = ndb

    return pl.pallas_call(
        body,
        grid=(_GRID,),
        in_specs=[
            pl.BlockSpec((2, _BN, _H), lambda i: (0, i, 0)),
            pl.BlockSpec((16, 128), lambda i: (i, 0)),
            pl.BlockSpec((16, 128), lambda i: (i + _GRID, 0)),
        ],
        out_specs=[
            pl.BlockSpec((_BN, _H), lambda i: (i, 0)),
            pl.BlockSpec((_BN, _H), lambda i: (i, 0)),
            pl.BlockSpec((_BN, _H), lambda i: (i, 0)),
            pl.BlockSpec((_BN, _H), lambda i: (i, 0)),
        ],
        out_shape=[
            jax.ShapeDtypeStruct((_NP, _H), _f32),
            jax.ShapeDtypeStruct((_NP, _H), _f32),
            jax.ShapeDtypeStruct((_NP, _H), _f32),
            jax.ShapeDtypeStruct((_NP, _H), _f32),
        ],
    )(xp, norms_lm, norms_lm)


def _tc_layer_pass1(aggp, nd, W, b):
    def body(agg_ref, nd_ref, w_ref, b_ref, y_ref, st_ref, acc):
        i = pl.program_id(0)

        @pl.when(i == 0)
        def _():
            acc[...] = jnp.zeros((8, _H), _f32)

        z = (agg_ref[0] + agg_ref[1]) * nd_ref[...]
        y = jnp.dot(z, w_ref[...], preferred_element_type=_f32) + b_ref[...]
        y_ref[...] = y
        rows = lax.broadcasted_iota(_i32, (_BN, 1), 0) + i * _BN
        mask = rows < _N
        ym = jnp.where(mask, y, 0.0)
        ym2 = jnp.where(mask, y * y, 0.0)
        acc[0:1, :] += jnp.sum(ym, axis=0, keepdims=True)
        acc[1:2, :] += jnp.sum(ym2, axis=0, keepdims=True)
        st_ref[...] = acc[...]

    return pl.pallas_call(
        body,
        grid=(_GRID,),
        in_specs=[
            pl.BlockSpec((2, _BN, _H), lambda i: (0, i, 0)),
            pl.BlockSpec((_BN, _H), lambda i: (i, 0)),
            pl.BlockSpec((_H, _H), lambda i: (0, 0)),
            pl.BlockSpec((1, _H), lambda i: (0, 0)),
        ],
        out_specs=[
            pl.BlockSpec((_BN, _H), lambda i: (i, 0)),
            pl.BlockSpec((8, _H), lambda i: (0, 0)),
        ],
        out_shape=[
            jax.ShapeDtypeStruct((_NP, _H), _f32),
            jax.ShapeDtypeStruct((8, _H), _f32),
        ],
        scratch_shapes=[pltpu.VMEM((8, _H), _f32)],
        compiler_params=pltpu.CompilerParams(
            dimension_semantics=("arbitrary",)),
    )(aggp, nd, W, b)


def _tc_layer_pass2(y, stats, gamma, beta, xprev, ns):
    def body(y_ref, st_ref, g_ref, be_ref, xp_ref, ns_ref, x_ref, xn_ref):
        mean = st_ref[0:1, :] * (1.0 / _N)
        var = st_ref[1:2, :] * (1.0 / _N) - mean * mean
        scale = g_ref[...] / jnp.sqrt(var + 1e-5)
        t = (y_ref[...] - mean) * scale + be_ref[...]
        t = jnp.maximum(t, 0.0)
        x = xp_ref[...] + t
        x_ref[...] = x
        xn_ref[...] = x * ns_ref[...]

    return pl.pallas_call(
        body,
        grid=(_GRID,),
        in_specs=[
            pl.BlockSpec((_BN, _H), lambda i: (i, 0)),
            pl.BlockSpec((8, _H), lambda i: (0, 0)),
            pl.BlockSpec((1, _H), lambda i: (0, 0)),
            pl.BlockSpec((1, _H), lambda i: (0, 0)),
            pl.BlockSpec((_BN, _H), lambda i: (i, 0)),
            pl.BlockSpec((_BN, _H), lambda i: (i, 0)),
        ],
        out_specs=[
            pl.BlockSpec((_BN, _H), lambda i: (i, 0)),
            pl.BlockSpec((_BN, _H), lambda i: (i, 0)),
        ],
        out_shape=[
            jax.ShapeDtypeStruct((_NP, _H), _f32),
            jax.ShapeDtypeStruct((_NP, _H), _f32),
        ],
    )(y, stats, gamma, beta, xprev, ns)


def _tc_readout(x, gid, W0, b0, W1, b1, W2, b2):
    def body(x_ref, g_ref, w0, b0r, w1, b1r, w2, b2r, z_ref, hg, cnt):
        i = pl.program_id(0)

        @pl.when(i == 0)
        def _():
            hg[...] = jnp.zeros((_NG, _H), _f32)
            cnt[...] = jnp.zeros((_NG, 1), _f32)

        cols = lax.broadcasted_iota(_i32, (_BN, _NG), 1)
        m = (g_ref[...] == cols).astype(_f32)
        hg[...] += lax.dot_general(m, x_ref[...], (((0,), (0,)), ((), ())),
                                   precision=lax.Precision.HIGHEST,
                                   preferred_element_type=_f32)
        cnt[...] += lax.dot_general(m, jnp.ones((_BN, 1), _f32),
                                    (((0,), (0,)), ((), ())),
                                    precision=lax.Precision.HIGHEST,
                                    preferred_element_type=_f32)

        @pl.when(i == _GRID - 1)
        def _():
            hgm = hg[...] / jnp.maximum(cnt[...], 1.0)
            z = jnp.maximum(jnp.dot(hgm, w0[...],
                                    preferred_element_type=_f32) + b0r[...], 0.0)
            z = jnp.maximum(jnp.dot(z, w1[...],
                                    preferred_element_type=_f32) + b1r[...], 0.0)
            z_ref[...] = jnp.dot(z, w2[...],
                                 preferred_element_type=_f32) + b2r[...]

    return pl.pallas_call(
        body,
        grid=(_GRID,),
        in_specs=[
            pl.BlockSpec((_BN, _H), lambda i: (i, 0)),
            pl.BlockSpec((_BN, 1), lambda i: (i, 0)),
            pl.BlockSpec((_H, _H // 2), lambda i: (0, 0)),
            pl.BlockSpec((1, _H // 2), lambda i: (0, 0)),
            pl.BlockSpec((_H // 2, _H // 4), lambda i: (0, 0)),
            pl.BlockSpec((1, _H // 4), lambda i: (0, 0)),
            pl.BlockSpec((_H // 4, 1), lambda i: (0, 0)),
            pl.BlockSpec((1, 1), lambda i: (0, 0)),
        ],
        out_specs=pl.BlockSpec((_NG, 1), lambda i: (0, 0)),
        out_shape=jax.ShapeDtypeStruct((_NG, 1), _f32),
        scratch_shapes=[pltpu.VMEM((_NG, _H), _f32),
                        pltpu.VMEM((_NG, 1), _f32)],
        compiler_params=pltpu.CompilerParams(
            dimension_semantics=("arbitrary",)),
    )(x, gid, W0, b0, W1, b1, W2, b2)


def kernel(h, edge_index, e, graph_ids, atom_emb, Ws, bs, gammas, betas,
           mlp_W0, mlp_b0, mlp_W1, mlp_b1, mlp_W2, mlp_b2):
    del e

    flat_table = atom_emb.reshape(_NF * 119, _H)
    enc_src = (h.astype(_i32) + 119 * jnp.arange(_NF, dtype=_i32)).reshape(-1)
    enc_dst = jnp.broadcast_to(jnp.arange(_N, dtype=_i32)[:, None],
                               (_N, _NF)).reshape(-1)
    enc_src3, enc_dst3 = _pad_edges(enc_src, enc_dst, _CE_E, 0)
    order = jnp.argsort(edge_index[1], stable=True)
    src3, dst3 = _pad_edges(edge_index[0][order], edge_index[1][order],
                            _CE_L, _DUMMY)

    gid = jnp.concatenate([graph_ids.astype(_i32),
                           jnp.full((_NP - _N,), -1, _i32)]).reshape(_NP, 1)
    b0 = mlp_b0.reshape(1, -1)
    b1 = mlp_b1.reshape(1, -1)
    b2 = mlp_b2.reshape(1, 1)

    sc_encode = _make_sc_scatter(_CE_E)
    sc_layer = _make_sc_scatter(_CE_L)
    sc_deg = _make_sc_degrees()

    xp = sc_encode(flat_table, enc_src3, enc_dst3)
    degp = sc_deg(src3, dst3)
    norms_lm = _tc_norms(degp.reshape(_NW, 2 * _NP // 128, 128))
    x, xn, ns, nd = _tc_pass0(xp, norms_lm)

    for l in range(_L):
        aggp = sc_layer(xn, src3, dst3)
        y, stats = _tc_layer_pass1(aggp, nd, Ws[l], bs[l].reshape(1, _H))
        x, xn = _tc_layer_pass2(y, stats, gammas[l].reshape(1, _H),
                                betas[l].reshape(1, _H), x, ns)

    return _tc_readout(x, gid, mlp_W0, b0, mlp_W1, b1, mlp_W2, b2)


# speedup vs baseline: 4.4060x; 1.0492x over previous
import functools

import jax
import jax.numpy as jnp
from jax import lax
from jax.experimental import pallas as pl
from jax.experimental.pallas import tpu as pltpu
from jax.experimental.pallas import tpu_sc as plsc

_N = 10000
_E = 320000
_H = 128
_L = 4
_NF = 9
_NG = 128

_NP = 10240
_NW = 32
_DUMMY = 10000
_CE_L = 79
_CE_E = 23
_BN = 2048
_GRID = _NP // _BN

_f32 = jnp.float32
_i32 = jnp.int32


def _pad_edges(src, dst, nchunks, src_pad):
    total = _NW * nchunks * 128
    pad = total - src.shape[0]
    src = jnp.concatenate([src.astype(_i32), jnp.full((pad,), src_pad, _i32)])
    dst = jnp.concatenate([dst.astype(_i32), jnp.full((pad,), _DUMMY, _i32)])
    src = src.reshape(_NW, 128, nchunks).swapaxes(1, 2)
    dst = dst.reshape(_NW, 128, nchunks).swapaxes(1, 2)
    return src, dst


def _make_sc_scatter(nchunks):
    mesh = plsc.VectorSubcoreMesh(core_axis_name="c", subcore_axis_name="s")

    @functools.partial(
        pl.kernel,
        out_type=jax.ShapeDtypeStruct((2, _NP, _H), _f32),
        mesh=mesh,
        scratch_types=[
            pltpu.VMEM((nchunks, 128), _i32),
            pltpu.VMEM((nchunks, 128), _i32),
            pltpu.VMEM((128, _H), _f32),
            pltpu.VMEM_SHARED((_NP, _H), _f32),
            pltpu.SemaphoreType.DMA,
        ],
    )
    def sc_scatter(table, src3, dst3, out, srcv, dstv, rows, agg_sh, sem):
        c = lax.axis_index("c")
        s = lax.axis_index("s")
        tid = c * 16 + s

        def zrow(i, _):
            for q in range(8):
                rows[i, pl.ds(q * 16, 16)] = jnp.zeros((16,), _f32)
            return 0

        lax.fori_loop(0, 128, zrow, 0)
        for k in range(_NP // (16 * 128)):
            pltpu.sync_copy(rows, agg_sh.at[pl.ds(s * 640 + k * 128, 128)])
        plsc.subcore_barrier()

        pltpu.sync_copy(src3.at[tid], srcv)
        pltpu.sync_copy(dst3.at[tid], dstv)

        def step(j, _):
            pltpu.async_copy(table.at[srcv.at[j]], rows, sem).wait()
            pltpu.sync_copy(rows, agg_sh.at[dstv.at[j]], add=True)
            return 0

        lax.fori_loop(0, nchunks, step, 0)
        plsc.subcore_barrier()
        pltpu.sync_copy(agg_sh.at[pl.ds(s * 640, 640)],
                        out.at[c, pl.ds(s * 640, 640)])

    return sc_scatter


def _make_sc_degrees():
    mesh = plsc.VectorSubcoreMesh(core_axis_name="c", subcore_axis_name="s")
    nbins = 2 * _NP

    @functools.partial(
        pl.kernel,
        out_type=jax.ShapeDtypeStruct((_NW, nbins), _f32),
        mesh=mesh,
        scratch_types=[
            pltpu.VMEM((_CE_L, 128), _i32),
            pltpu.VMEM((_CE_L, 128), _i32),
            pltpu.VMEM((nbins,), _f32),
        ],
        compiler_params=pltpu.CompilerParams(needs_layout_passes=False),
    )
    def sc_deg(src3, dst3, out, srcv, dstv, degv):
        c = lax.axis_index("c")
        s = lax.axis_index("s")
        tid = c * 16 + s

        def zrow(i, _):
            degv[pl.ds(i * 16, 16)] = jnp.zeros((16,), _f32)
            return 0

        lax.fori_loop(0, nbins // 16, zrow, 0)
        pltpu.sync_copy(src3.at[tid], srcv)
        pltpu.sync_copy(dst3.at[tid], dstv)
        ones = jnp.ones((16,), _f32)

        def step(j, _):
            for q in range(8):
                a = srcv[j, pl.ds(q * 16, 16)]
                plsc.addupdate_scatter(degv, [a], ones)
                b = dstv[j, pl.ds(q * 16, 16)] + _NP
                plsc.addupdate_scatter(degv, [b], ones)
            return 0

        lax.fori_loop(0, _CE_L, step, 0)
        pltpu.sync_copy(degv, out.at[tid])

    return sc_deg


def _tc_norms(degp):
    def body(d_ref, n_ref):
        d = jnp.sum(d_ref[...], axis=0)
        n_ref[...] = 1.0 / jnp.sqrt(jnp.maximum(d, 1.0))

    nrows = 2 * _NP // 128
    return pl.pallas_call(
        body,
        out_shape=jax.ShapeDtypeStruct((nrows, 128), _f32),
    )(degp)


def _expand_lanes(lm):
    r = lax.broadcasted_iota(_i32, (128, 128), 0)
    c = lax.broadcasted_iota(_i32, (128, 128), 1)
    ident = (r == c).astype(_f32)
    parts = []
    for k in range(16):
        col = lax.dot_general(ident, lm[k:k + 1, :], (((1,), (1,)), ((), ())),
                              precision=lax.Precision.HIGHEST,
                              preferred_element_type=_f32)
        parts.append(col * jnp.ones((1, _H), _f32))
    return jnp.concatenate(parts, axis=0)


def _tc_pass0(xp, norms_lm):
    def body(xp_ref, ns_ref, nd_ref, x_ref, xn_ref, nsb_ref, ndb_ref):
        nsb = _expand_lanes(ns_ref[...])
        ndb = _expand_lanes(nd_ref[...])
        x = xp_ref[0] + xp_ref[1]
        x_ref[...] = x
        xn_ref[...] = x * nsb
        nsb_ref[...] = nsb
        ndb_ref[...] = ndb

    return pl.pallas_call(
        body,
        grid=(_GRID,),
        in_specs=[
            pl.BlockSpec((2, _BN, _H), lambda i: (0, i, 0)),
            pl.BlockSpec((16, 128), lambda i: (i, 0)),
            pl.BlockSpec((16, 128), lambda i: (i + _GRID, 0)),
        ],
        out_specs=[
            pl.BlockSpec((_BN, _H), lambda i: (i, 0)),
            pl.BlockSpec((_BN, _H), lambda i: (i, 0)),
            pl.BlockSpec((_BN, _H), lambda i: (i, 0)),
            pl.BlockSpec((_BN, _H), lambda i: (i, 0)),
        ],
        out_shape=[
            jax.ShapeDtypeStruct((_NP, _H), _f32),
            jax.ShapeDtypeStruct((_NP, _H), _f32),
            jax.ShapeDtypeStruct((_NP, _H), _f32),
            jax.ShapeDtypeStruct((_NP, _H), _f32),
        ],
    )(xp, norms_lm, norms_lm)


def _tc_layer_pass1(aggp, nd, W, b):
    def body(agg_ref, nd_ref, w_ref, b_ref, y_ref, st_ref, acc):
        i = pl.program_id(0)

        @pl.when(i == 0)
        def _():
            acc[...] = jnp.zeros((8, _H), _f32)

        z = (agg_ref[0] + agg_ref[1]) * nd_ref[...]
        y = jnp.dot(z, w_ref[...], preferred_element_type=_f32) + b_ref[...]
        y_ref[...] = y
        rows = lax.broadcasted_iota(_i32, (_BN, 1), 0) + i * _BN
        mask = rows < _N
        ym = jnp.where(mask, y, 0.0)
        ym2 = jnp.where(mask, y * y, 0.0)
        acc[0:1, :] += jnp.sum(ym, axis=0, keepdims=True)
        acc[1:2, :] += jnp.sum(ym2, axis=0, keepdims=True)
        st_ref[...] = acc[...]

    return pl.pallas_call(
        body,
        grid=(_GRID,),
        in_specs=[
            pl.BlockSpec((2, _BN, _H), lambda i: (0, i, 0)),
            pl.BlockSpec((_BN, _H), lambda i: (i, 0)),
            pl.BlockSpec((_H, _H), lambda i: (0, 0)),
            pl.BlockSpec((1, _H), lambda i: (0, 0)),
        ],
        out_specs=[
            pl.BlockSpec((_BN, _H), lambda i: (i, 0)),
            pl.BlockSpec((8, _H), lambda i: (0, 0)),
        ],
        out_shape=[
            jax.ShapeDtypeStruct((_NP, _H), _f32),
            jax.ShapeDtypeStruct((8, _H), _f32),
        ],
        scratch_shapes=[pltpu.VMEM((8, _H), _f32)],
        compiler_params=pltpu.CompilerParams(
            dimension_semantics=("arbitrary",)),
    )(aggp, nd, W, b)


def _tc_layer_pass2(y, stats, gamma, beta, xprev, ns):
    def body(y_ref, st_ref, g_ref, be_ref, xp_ref, ns_ref, x_ref, xn_ref):
        mean = st_ref[0:1, :] * (1.0 / _N)
        var = st_ref[1:2, :] * (1.0 / _N) - mean * mean
        scale = g_ref[...] / jnp.sqrt(var + 1e-5)
        t = (y_ref[...] - mean) * scale + be_ref[...]
        t = jnp.maximum(t, 0.0)
        x = xp_ref[...] + t
        x_ref[...] = x
        xn_ref[...] = x * ns_ref[...]

    return pl.pallas_call(
        body,
        grid=(_GRID,),
        in_specs=[
            pl.BlockSpec((_BN, _H), lambda i: (i, 0)),
            pl.BlockSpec((8, _H), lambda i: (0, 0)),
            pl.BlockSpec((1, _H), lambda i: (0, 0)),
            pl.BlockSpec((1, _H), lambda i: (0, 0)),
            pl.BlockSpec((_BN, _H), lambda i: (i, 0)),
            pl.BlockSpec((_BN, _H), lambda i: (i, 0)),
        ],
        out_specs=[
            pl.BlockSpec((_BN, _H), lambda i: (i, 0)),
            pl.BlockSpec((_BN, _H), lambda i: (i, 0)),
        ],
        out_shape=[
            jax.ShapeDtypeStruct((_NP, _H), _f32),
            jax.ShapeDtypeStruct((_NP, _H), _f32),
        ],
    )(y, stats, gamma, beta, xprev, ns)


def _tc_readout(x, gid, W0, b0, W1, b1, W2, b2):
    def body(x_ref, g_ref, w0, b0r, w1, b1r, w2, b2r, z_ref, hg, cnt):
        i = pl.program_id(0)

        @pl.when(i == 0)
        def _():
            hg[...] = jnp.zeros((_NG, _H), _f32)
            cnt[...] = jnp.zeros((_NG, 1), _f32)

        cols = lax.broadcasted_iota(_i32, (_BN, _NG), 1)
        m = (g_ref[...] == cols).astype(_f32)
        hg[...] += lax.dot_general(m, x_ref[...], (((0,), (0,)), ((), ())),
                                   precision=lax.Precision.HIGHEST,
                                   preferred_element_type=_f32)
        cnt[...] += lax.dot_general(m, jnp.ones((_BN, 1), _f32),
                                    (((0,), (0,)), ((), ())),
                                    precision=lax.Precision.HIGHEST,
                                    preferred_element_type=_f32)

        @pl.when(i == _GRID - 1)
        def _():
            hgm = hg[...] / jnp.maximum(cnt[...], 1.0)
            z = jnp.maximum(jnp.dot(hgm, w0[...],
                                    preferred_element_type=_f32) + b0r[...], 0.0)
            z = jnp.maximum(jnp.dot(z, w1[...],
                                    preferred_element_type=_f32) + b1r[...], 0.0)
            z_ref[...] = jnp.dot(z, w2[...],
                                 preferred_element_type=_f32) + b2r[...]

    return pl.pallas_call(
        body,
        grid=(_GRID,),
        in_specs=[
            pl.BlockSpec((_BN, _H), lambda i: (i, 0)),
            pl.BlockSpec((_BN, 1), lambda i: (i, 0)),
            pl.BlockSpec((_H, _H // 2), lambda i: (0, 0)),
            pl.BlockSpec((1, _H // 2), lambda i: (0, 0)),
            pl.BlockSpec((_H // 2, _H // 4), lambda i: (0, 0)),
            pl.BlockSpec((1, _H // 4), lambda i: (0, 0)),
            pl.BlockSpec((_H // 4, 1), lambda i: (0, 0)),
            pl.BlockSpec((1, 1), lambda i: (0, 0)),
        ],
        out_specs=pl.BlockSpec((_NG, 1), lambda i: (0, 0)),
        out_shape=jax.ShapeDtypeStruct((_NG, 1), _f32),
        scratch_shapes=[pltpu.VMEM((_NG, _H), _f32),
                        pltpu.VMEM((_NG, 1), _f32)],
        compiler_params=pltpu.CompilerParams(
            dimension_semantics=("arbitrary",)),
    )(x, gid, W0, b0, W1, b1, W2, b2)


def kernel(h, edge_index, e, graph_ids, atom_emb, Ws, bs, gammas, betas,
           mlp_W0, mlp_b0, mlp_W1, mlp_b1, mlp_W2, mlp_b2):
    del e

    flat_table = atom_emb.reshape(_NF * 119, _H)
    enc_src = (h.astype(_i32) + 119 * jnp.arange(_NF, dtype=_i32)).reshape(-1)
    enc_dst = jnp.broadcast_to(jnp.arange(_N, dtype=_i32)[:, None],
                               (_N, _NF)).reshape(-1)
    enc_src3, enc_dst3 = _pad_edges(enc_src, enc_dst, _CE_E, 0)
    order = jnp.argsort(edge_index[1], stable=True)
    src3, dst3 = _pad_edges(edge_index[0][order], edge_index[1][order],
                            _CE_L, _DUMMY)

    gid = jnp.concatenate([graph_ids.astype(_i32),
                           jnp.full((_NP - _N,), -1, _i32)]).reshape(_NP, 1)
    b0 = mlp_b0.reshape(1, -1)
    b1 = mlp_b1.reshape(1, -1)
    b2 = mlp_b2.reshape(1, 1)

    sc_encode = _make_sc_scatter(_CE_E)
    sc_layer = _make_sc_scatter(_CE_L)
    sc_deg = _make_sc_degrees()

    xp = sc_encode(flat_table, enc_src3, enc_dst3)
    degp = sc_deg(src3, dst3)
    norms_lm = _tc_norms(degp.reshape(_NW, 2 * _NP // 128, 128))
    x, xn, ns, nd = _tc_pass0(xp, norms_lm)

    for l in range(_L):
        aggp = sc_layer(xn, src3, dst3)
        y, stats = _tc_layer_pass1(aggp, nd, Ws[l], bs[l].reshape(1, _H))
        x, xn = _tc_layer_pass2(y, stats, gammas[l].reshape(1, _H),
                                betas[l].reshape(1, _H), x, ns)

    return _tc_readout(x, gid, mlp_W0, b0, mlp_W1, b1, mlp_W2, b2)
